# Initial kernel scaffold; baseline (speedup 1.0000x reference)
import functools

import jax
import jax.numpy as jnp
from jax import lax
from jax.experimental import pallas as pl
from jax.experimental.pallas import tpu as pltpu
from jax.experimental.pallas import tpu_sc as plsc

_NUM_SCALES = 16
_DIM = 64
_LANES = 16
_GROUPS = _DIM // _LANES

_NC = 2
_NS = 16
_NW = _NC * _NS

_CHUNK = 512


def _sc_body(x_hbm, p_hbm, emb_hbm, out_hbm, buf, idxbuf, table):
    n_rows = x_hbm.shape[0]
    rows_per_w = n_rows // _NW
    n_chunks = rows_per_w // _CHUNK

    wid = lax.axis_index("s") * _NC + lax.axis_index("c")
    w_base = wid * rows_per_w

    pltpu.sync_copy(emb_hbm, table)

    def chunk_body(g, carry):
        start = w_base + g * _CHUNK
        pltpu.sync_copy(x_hbm.at[pl.ds(start, _CHUNK)], buf)
        pltpu.sync_copy(p_hbm.at[pl.ds(start, _CHUNK)], idxbuf)

        def row_body(r, rcarry):
            p = idxbuf[r]
            for q in range(_GROUPS):
                sl = pl.ds(q * _LANES, _LANES)
                buf[r, sl] = buf[r, sl] + table[p, sl]
            return rcarry

        lax.fori_loop(0, _CHUNK, row_body, 0)
        pltpu.sync_copy(buf, out_hbm.at[pl.ds(start, _CHUNK)])
        return carry

    lax.fori_loop(0, n_chunks, chunk_body, 0)


def kernel(inputs, inputs_scale_positions, scale_emb):
    b, l, d = inputs.shape
    n = b * l
    x = inputs.reshape(n, d)
    p = inputs_scale_positions.reshape(n)

    mesh = plsc.VectorSubcoreMesh(core_axis_name="c", subcore_axis_name="s")
    run = pl.kernel(
        _sc_body,
        mesh=mesh,
        out_type=jax.ShapeDtypeStruct((n, d), jnp.float32),
        scratch_types=[
            pltpu.VMEM((_CHUNK, d), jnp.float32),
            pltpu.VMEM((_CHUNK,), jnp.int32),
            pltpu.VMEM((_NUM_SCALES, d), jnp.float32),
        ],
    )
    out = run(x, p, scale_emb)
    return out.reshape(b, l, d)


# speedup vs baseline: 2.6154x; 2.6154x over previous
import functools

import jax
import jax.numpy as jnp
from jax import lax
from jax.experimental import pallas as pl
from jax.experimental.pallas import tpu as pltpu
from jax.experimental.pallas import tpu_sc as plsc

_NUM_SCALES = 16
_DIM = 64
_LANES = 16
_GROUPS = _DIM // _LANES

_NC = 2
_NS = 16
_NW = _NC * _NS

_CHUNK = 512


def _sc_body(x_hbm, p_hbm, emb_hbm, out_hbm, buf, idxbuf, table):
    n_rows = x_hbm.shape[0]
    rows_per_w = n_rows // _NW
    n_chunks = rows_per_w // _CHUNK

    wid = lax.axis_index("s") * _NC + lax.axis_index("c")
    w_base = wid * rows_per_w

    pltpu.sync_copy(emb_hbm, table)

    def chunk_body(g, carry):
        start = w_base + g * _CHUNK
        pltpu.sync_copy(x_hbm.at[pl.ds(start, _CHUNK)], buf)
        pltpu.sync_copy(p_hbm.at[pl.ds(start, _CHUNK)], idxbuf)

        def row_body(rb, rcarry):
            r0 = rb * _LANES
            pvec = idxbuf[pl.ds(r0, _LANES)]
            for i in range(_LANES):
                p = pvec[i]
                for q in range(_GROUPS):
                    sl = pl.ds(q * _LANES, _LANES)
                    buf[r0 + i, sl] = buf[r0 + i, sl] + table[p, sl]
            return rcarry

        lax.fori_loop(0, _CHUNK // _LANES, row_body, 0)
        pltpu.sync_copy(buf, out_hbm.at[pl.ds(start, _CHUNK)])
        return carry

    lax.fori_loop(0, n_chunks, chunk_body, 0)


def kernel(inputs, inputs_scale_positions, scale_emb):
    b, l, d = inputs.shape
    n = b * l
    x = inputs.reshape(n, d)
    p = inputs_scale_positions.reshape(n)

    mesh = plsc.VectorSubcoreMesh(core_axis_name="c", subcore_axis_name="s")
    run = pl.kernel(
        _sc_body,
        mesh=mesh,
        out_type=jax.ShapeDtypeStruct((n, d), jnp.float32),
        scratch_types=[
            pltpu.VMEM((_CHUNK, d), jnp.float32),
            pltpu.VMEM((_CHUNK,), jnp.int32),
            pltpu.VMEM((_NUM_SCALES, d), jnp.float32),
        ],
    )
    out = run(x, p, scale_emb)
    return out.reshape(b, l, d)


# speedup vs baseline: 3.7986x; 1.4524x over previous
import functools

import jax
import jax.numpy as jnp
from jax import lax
from jax.experimental import pallas as pl
from jax.experimental.pallas import tpu as pltpu
from jax.experimental.pallas import tpu_sc as plsc

_NUM_SCALES = 16
_DIM = 64
_LANES = 16
_GROUPS = _DIM // _LANES

_NC = 2
_NS = 16
_NW = _NC * _NS

_CHUNK = 512


def _sc_body(x_hbm, p_hbm, emb_hbm, out_hbm, buf, idxbuf, table):
    n_rows = x_hbm.shape[0]
    rows_per_w = n_rows // _NW
    n_chunks = rows_per_w // _CHUNK

    wid = lax.axis_index("s") * _NC + lax.axis_index("c")
    w_base = wid * rows_per_w

    pltpu.sync_copy(emb_hbm, table)

    def chunk_body(g, carry):
        start = w_base + g * _CHUNK
        pltpu.sync_copy(x_hbm.at[pl.ds(start, _CHUNK)], buf)
        pltpu.sync_copy(p_hbm.at[pl.ds(start, _CHUNK)], idxbuf)

        @plsc.parallel_loop(0, _CHUNK // _LANES, unroll=1)
        def row_body(rb):
            r0 = rb * _LANES
            pvec = idxbuf[pl.ds(r0, _LANES)]
            for i in range(_LANES):
                p = pvec[i]
                ins = [buf[r0 + i, pl.ds(q * _LANES, _LANES)]
                       for q in range(_GROUPS)]
                embs = [table[p, pl.ds(q * _LANES, _LANES)]
                        for q in range(_GROUPS)]
                sums = [a + b for a, b in zip(ins, embs)]
                for q in range(_GROUPS):
                    buf[r0 + i, pl.ds(q * _LANES, _LANES)] = sums[q]
        pltpu.sync_copy(buf, out_hbm.at[pl.ds(start, _CHUNK)])
        return carry

    lax.fori_loop(0, n_chunks, chunk_body, 0)


def kernel(inputs, inputs_scale_positions, scale_emb):
    b, l, d = inputs.shape
    n = b * l
    x = inputs.reshape(n, d)
    p = inputs_scale_positions.reshape(n)

    mesh = plsc.VectorSubcoreMesh(core_axis_name="c", subcore_axis_name="s")
    run = pl.kernel(
        _sc_body,
        mesh=mesh,
        out_type=jax.ShapeDtypeStruct((n, d), jnp.float32),
        scratch_types=[
            pltpu.VMEM((_CHUNK, d), jnp.float32),
            pltpu.VMEM((_CHUNK,), jnp.int32),
            pltpu.VMEM((_NUM_SCALES, d), jnp.float32),
        ],
    )
    out = run(x, p, scale_emb)
    return out.reshape(b, l, d)
